# baseline (speedup 1.0000x reference)
import functools

import jax
import jax.numpy as jnp
from jax import lax
from jax.experimental import pallas as pl
from jax.experimental.pallas import tpu as pltpu
from jax.experimental.pallas import tpu_sc as plsc

NUM_HEADS = 16
N = 2048
NN = N * N

_info = plsc.get_sparse_core_info()
_NC, _NS, _L = _info.num_cores, _info.num_subcores, _info.num_lanes
_NW = _NC * _NS
_B_PER_W = NN // _NW
_C = 1024
_CHUNKS = _B_PER_W // _C
_TPAD = _C + 1
_CPB = 16
_IBC = _CPB * _C
_NBLK = _CHUNKS // _CPB


def _sc_gather_transpose(idx_flat, table):
    mesh = plsc.VectorSubcoreMesh(core_axis_name="c", subcore_axis_name="s")

    @functools.partial(
        pl.kernel,
        mesh=mesh,
        compiler_params=pltpu.CompilerParams(
            use_tc_tiling_on_sc=False, needs_layout_passes=False
        ),
        out_type=jax.ShapeDtypeStruct((NUM_HEADS, NN), jnp.float32),
        scratch_types=[
            pltpu.VMEM((_C,), jnp.int32),
            pltpu.VMEM((_C,), jnp.int32),
            pltpu.VMEM((_C,), jnp.int32),
            pltpu.VMEM((_C,), jnp.int32),
            pltpu.VMEM((_C, NUM_HEADS), jnp.float32),
            pltpu.VMEM((_C, NUM_HEADS), jnp.float32),
            pltpu.VMEM((_C, NUM_HEADS), jnp.float32),
            pltpu.VMEM((_C, NUM_HEADS), jnp.float32),
            pltpu.VMEM((NUM_HEADS, _TPAD), jnp.float32),
            pltpu.VMEM((NUM_HEADS, _TPAD), jnp.float32),
            pltpu.SemaphoreType.DMA,
            pltpu.SemaphoreType.DMA,
            pltpu.SemaphoreType.DMA,
            pltpu.SemaphoreType.DMA,
            pltpu.SemaphoreType.DMA,
            pltpu.SemaphoreType.DMA,
            pltpu.SemaphoreType.DMA,
            pltpu.SemaphoreType.DMA,
            pltpu.SemaphoreType.DMA,
            pltpu.SemaphoreType.DMA,
        ],
    )
    def k(table_hbm, idx_hbm, out_hbm, i0, i1, i2, i3, r0, r1, r2, r3,
          t0, t1, is0, is1, is2, is3, g0, g1, g2, g3, w0, w1):
        idx_v = (i0, i1, i2, i3)
        rows_v = (r0, r1, r2, r3)
        trans_v = (t0, t1)
        isem = (is0, is1, is2, is3)
        gsem = (g0, g1, g2, g3)
        wsem = (w0, w1)
        wid = lax.axis_index("s") * _NC + lax.axis_index("c")
        wstart = wid * _B_PER_W
        lane = lax.broadcasted_iota(jnp.int32, (_L,), 0)

        def idx_desc(t, p):
            return pltpu.make_async_copy(
                idx_hbm.at[pl.ds(wstart + t * _C, _C)], idx_v[p], isem[p]
            )

        def gather_desc(p):
            return pltpu.make_async_copy(
                table_hbm.at[idx_v[p]], rows_v[p], gsem[p]
            )

        def wb_desc(t, p):
            return pltpu.make_async_copy(
                trans_v[p].at[:, pl.ds(0, _C)],
                out_hbm.at[:, pl.ds(wstart + t * _C, _C)],
                wsem[p],
            )

        def compute_chunk(p, tp):
            rref = rows_v[p]
            tref = trans_v[tp]

            def kb_body(kb, c3):
                base_k = kb * _L
                colbase = jnp.full((_L,), base_k, jnp.int32)
                for j in range(_L):
                    v = rref[base_k + j]
                    plsc.store_scatter(tref, [lane, colbase + j], v)
                return c3

            lax.fori_loop(0, _C // _L, kb_body, 0)

        for p in range(4):
            idx_desc(p, p).start()
        for p in range(3):
            idx_desc(p, p).wait()
            gather_desc(p).start()

        def quad_body(tt, carry):
            for cb in range(4):
                t = tt * 4 + cb
                tp = cb % 2
                np3 = (cb + 3) % 4

                gather_desc(cb).wait()

                @pl.when(t + 4 < _CHUNKS)
                def _():
                    idx_desc(t + 4, cb).start()

                @pl.when(t + 3 < _CHUNKS)
                def _():
                    idx_desc(t + 3, np3).wait()
                    gather_desc(np3).start()

                @pl.when(t >= 2)
                def _():
                    wb_desc(t - 2, tp).wait()

                compute_chunk(cb, tp)
                wb_desc(t, tp).start()
            return carry

        lax.fori_loop(0, _CHUNKS // 4, quad_body, 0)

        for b in range(2):
            wb_desc(_CHUNKS - 2 + b, b).wait()

    return k(table, idx_flat)


_TI = N // 8
_TJ = N // 128


def _tc_body(g_ref, b_ref, out_ref):
    out_ref[0] = jnp.reshape(g_ref[0, :, 0, :, :], (N, 128)) + b_ref[0]


def _tc_assemble_add(g_raw, attn_bias):
    g5 = g_raw.reshape(NUM_HEADS, _TI, _TJ, 8, 128)
    return pl.pallas_call(
        _tc_body,
        grid=(_TJ, NUM_HEADS),
        in_specs=[
            pl.BlockSpec((1, _TI, 1, 8, 128), lambda tj, h: (h, 0, tj, 0, 0)),
            pl.BlockSpec((1, N, 128), lambda tj, h: (0, 0, tj)),
        ],
        out_specs=pl.BlockSpec((1, N, 128), lambda tj, h: (h, 0, tj)),
        out_shape=jax.ShapeDtypeStruct((NUM_HEADS, N, N), jnp.float32),
    )(g5, attn_bias)


def kernel(attn_bias, spatial_pos, W):
    idx_tile = (
        spatial_pos.reshape(_TI, 8, _TJ, 128)
        .transpose(0, 2, 1, 3)
        .reshape(NN)
    )
    g_raw = _sc_gather_transpose(idx_tile, W)
    out = _tc_assemble_add(g_raw, attn_bias)
    return out.reshape(1, NUM_HEADS, N, N)


# speedup vs baseline: 1.0048x; 1.0048x over previous
import functools

import jax
import jax.numpy as jnp
from jax import lax
from jax.experimental import pallas as pl
from jax.experimental.pallas import tpu as pltpu
from jax.experimental.pallas import tpu_sc as plsc

NUM_HEADS = 16
N = 2048
NN = N * N

_info = plsc.get_sparse_core_info()
_NC, _NS, _L = _info.num_cores, _info.num_subcores, _info.num_lanes
_NW = _NC * _NS
_B_PER_W = NN // _NW
_C = 1024
_CHUNKS = _B_PER_W // _C
_TPAD = _C + 1
_CPB = 16
_IBC = _CPB * _C
_NBLK = _CHUNKS // _CPB
_UNROLL = 32


def _sc_gather_transpose(idx_flat, table):
    mesh = plsc.VectorSubcoreMesh(core_axis_name="c", subcore_axis_name="s")

    @functools.partial(
        pl.kernel,
        mesh=mesh,
        compiler_params=pltpu.CompilerParams(
            use_tc_tiling_on_sc=False, needs_layout_passes=False
        ),
        out_type=jax.ShapeDtypeStruct((NUM_HEADS, NN), jnp.float32),
        scratch_types=[
            pltpu.VMEM((_C,), jnp.int32),
            pltpu.VMEM((_C,), jnp.int32),
            pltpu.VMEM((_C,), jnp.int32),
            pltpu.VMEM((_C,), jnp.int32),
            pltpu.VMEM((_C, NUM_HEADS), jnp.float32),
            pltpu.VMEM((_C, NUM_HEADS), jnp.float32),
            pltpu.VMEM((_C, NUM_HEADS), jnp.float32),
            pltpu.VMEM((_C, NUM_HEADS), jnp.float32),
            pltpu.VMEM((NUM_HEADS, _TPAD), jnp.float32),
            pltpu.VMEM((NUM_HEADS, _TPAD), jnp.float32),
            pltpu.SemaphoreType.DMA,
            pltpu.SemaphoreType.DMA,
            pltpu.SemaphoreType.DMA,
            pltpu.SemaphoreType.DMA,
            pltpu.SemaphoreType.DMA,
            pltpu.SemaphoreType.DMA,
            pltpu.SemaphoreType.DMA,
            pltpu.SemaphoreType.DMA,
            pltpu.SemaphoreType.DMA,
            pltpu.SemaphoreType.DMA,
        ],
    )
    def k(table_hbm, idx_hbm, out_hbm, i0, i1, i2, i3, r0, r1, r2, r3,
          t0, t1, is0, is1, is2, is3, g0, g1, g2, g3, w0, w1):
        idx_v = (i0, i1, i2, i3)
        rows_v = (r0, r1, r2, r3)
        trans_v = (t0, t1)
        isem = (is0, is1, is2, is3)
        gsem = (g0, g1, g2, g3)
        wsem = (w0, w1)
        wid = lax.axis_index("s") * _NC + lax.axis_index("c")
        wstart = wid * _B_PER_W
        lane = lax.broadcasted_iota(jnp.int32, (_L,), 0)

        def idx_desc(t, p):
            return pltpu.make_async_copy(
                idx_hbm.at[pl.ds(wstart + t * _C, _C)], idx_v[p], isem[p]
            )

        def gather_desc(p):
            return pltpu.make_async_copy(
                table_hbm.at[idx_v[p]], rows_v[p], gsem[p]
            )

        def wb_desc(t, p):
            return pltpu.make_async_copy(
                trans_v[p].at[:, pl.ds(0, _C)],
                out_hbm.at[:, pl.ds(wstart + t * _C, _C)],
                wsem[p],
            )

        def compute_chunk(p, tp):
            rref = rows_v[p]
            tref = trans_v[tp]

            def kb_body(kb, c3):
                base_k = kb * _UNROLL
                colbase = jnp.full((_L,), base_k, jnp.int32)
                for j in range(_UNROLL):
                    v = rref[base_k + j]
                    plsc.store_scatter(tref, [lane, colbase + j], v)
                return c3

            lax.fori_loop(0, _C // _UNROLL, kb_body, 0)

        for p in range(4):
            idx_desc(p, p).start()
        for p in range(3):
            idx_desc(p, p).wait()
            gather_desc(p).start()

        def quad_body(tt, carry):
            for cb in range(4):
                t = tt * 4 + cb
                tp = cb % 2
                np3 = (cb + 3) % 4

                gather_desc(cb).wait()

                @pl.when(t + 4 < _CHUNKS)
                def _():
                    idx_desc(t + 4, cb).start()

                @pl.when(t + 3 < _CHUNKS)
                def _():
                    idx_desc(t + 3, np3).wait()
                    gather_desc(np3).start()

                @pl.when(t >= 2)
                def _():
                    wb_desc(t - 2, tp).wait()

                compute_chunk(cb, tp)
                wb_desc(t, tp).start()
            return carry

        lax.fori_loop(0, _CHUNKS // 4, quad_body, 0)

        for b in range(2):
            wb_desc(_CHUNKS - 2 + b, b).wait()

    return k(table, idx_flat)


_TI = N // 8
_TJ = N // 128


def _tc_body(g_ref, b_ref, out_ref):
    out_ref[0] = jnp.reshape(g_ref[0, :, 0, :, :], (N, 128)) + b_ref[0]


def _tc_assemble_add(g_raw, attn_bias):
    g5 = g_raw.reshape(NUM_HEADS, _TI, _TJ, 8, 128)
    return pl.pallas_call(
        _tc_body,
        grid=(_TJ, NUM_HEADS),
        in_specs=[
            pl.BlockSpec((1, _TI, 1, 8, 128), lambda tj, h: (h, 0, tj, 0, 0)),
            pl.BlockSpec((1, N, 128), lambda tj, h: (0, 0, tj)),
        ],
        out_specs=pl.BlockSpec((1, N, 128), lambda tj, h: (h, 0, tj)),
        out_shape=jax.ShapeDtypeStruct((NUM_HEADS, N, N), jnp.float32),
    )(g5, attn_bias)


def kernel(attn_bias, spatial_pos, W):
    idx_tile = (
        spatial_pos.reshape(_TI, 8, _TJ, 128)
        .transpose(0, 2, 1, 3)
        .reshape(NN)
    )
    g_raw = _sc_gather_transpose(idx_tile, W)
    out = _tc_assemble_add(g_raw, attn_bias)
    return out.reshape(1, NUM_HEADS, N, N)


# speedup vs baseline: 1.0987x; 1.0935x over previous
import functools

import jax
import jax.numpy as jnp
from jax import lax
from jax.experimental import pallas as pl
from jax.experimental.pallas import tpu as pltpu
from jax.experimental.pallas import tpu_sc as plsc

NUM_HEADS = 16
N = 2048
NN = N * N

_info = plsc.get_sparse_core_info()
_NC, _NS, _L = _info.num_cores, _info.num_subcores, _info.num_lanes
_NW = _NC * _NS
_B_PER_W = NN // _NW
_C = 1024
_CHUNKS = _B_PER_W // _C
_TPAD = _C + 1
_CPB = 16
_IBC = _CPB * _C
_NBLK = _CHUNKS // _CPB
_UNROLL = 32


def _sc_gather_transpose(idx_flat, table):
    mesh = plsc.VectorSubcoreMesh(core_axis_name="c", subcore_axis_name="s")

    @functools.partial(
        pl.kernel,
        mesh=mesh,
        compiler_params=pltpu.CompilerParams(
            use_tc_tiling_on_sc=False, needs_layout_passes=False
        ),
        out_type=jax.ShapeDtypeStruct((NUM_HEADS, NN), jnp.float32),
        scratch_types=[
            pltpu.VMEM((_C,), jnp.int32),
            pltpu.VMEM((_C,), jnp.int32),
            pltpu.VMEM((_C,), jnp.int32),
            pltpu.VMEM((_C,), jnp.int32),
            pltpu.VMEM((_C, NUM_HEADS), jnp.float32),
            pltpu.VMEM((_C, NUM_HEADS), jnp.float32),
            pltpu.VMEM((_C, NUM_HEADS), jnp.float32),
            pltpu.VMEM((_C, NUM_HEADS), jnp.float32),
            pltpu.VMEM((NUM_HEADS, _TPAD), jnp.float32),
            pltpu.VMEM((NUM_HEADS, _TPAD), jnp.float32),
            pltpu.SemaphoreType.DMA,
            pltpu.SemaphoreType.DMA,
            pltpu.SemaphoreType.DMA,
            pltpu.SemaphoreType.DMA,
            pltpu.SemaphoreType.DMA,
            pltpu.SemaphoreType.DMA,
            pltpu.SemaphoreType.DMA,
            pltpu.SemaphoreType.DMA,
            pltpu.SemaphoreType.DMA,
            pltpu.SemaphoreType.DMA,
        ],
    )
    def k(table_hbm, idx_hbm, out_hbm, i0, i1, i2, i3, r0, r1, r2, r3,
          t0, t1, is0, is1, is2, is3, g0, g1, g2, g3, w0, w1):
        idx_v = (i0, i1, i2, i3)
        rows_v = (r0, r1, r2, r3)
        trans_v = (t0, t1)
        isem = (is0, is1, is2, is3)
        gsem = (g0, g1, g2, g3)
        wsem = (w0, w1)
        wid = lax.axis_index("s") * _NC + lax.axis_index("c")
        wstart = wid * _B_PER_W
        lane = lax.broadcasted_iota(jnp.int32, (_L,), 0)

        def idx_desc(t, p):
            return pltpu.make_async_copy(
                idx_hbm.at[pl.ds(wstart + t * _C, _C)], idx_v[p], isem[p]
            )

        def gather_desc(p):
            return pltpu.make_async_copy(
                table_hbm.at[idx_v[p]], rows_v[p], gsem[p]
            )

        def wb_desc(t, p):
            return pltpu.make_async_copy(
                trans_v[p].at[:, pl.ds(0, _C)],
                out_hbm.at[:, pl.ds(wstart + t * _C, _C)],
                wsem[p],
            )

        def compute_chunk(p, tp):
            rref = rows_v[p]
            tref = trans_v[tp]

            def kb_body(kb, c3):
                base_k = kb * _UNROLL
                colbase = jnp.full((_L,), base_k, jnp.int32)
                for j in range(_UNROLL):
                    v = rref[base_k + j]
                    plsc.store_scatter(tref, [lane, colbase + j], v)
                return c3

            lax.fori_loop(0, _C // _UNROLL, kb_body, 0)

        for p in range(4):
            idx_desc(p, p).start()
        for p in range(3):
            idx_desc(p, p).wait()
            gather_desc(p).start()

        def quad_body(tt, carry):
            for cb in range(4):
                t = tt * 4 + cb
                tp = cb % 2
                np3 = (cb + 3) % 4

                gather_desc(cb).wait()

                @pl.when(t + 4 < _CHUNKS)
                def _():
                    idx_desc(t + 4, cb).start()

                @pl.when(t + 3 < _CHUNKS)
                def _():
                    idx_desc(t + 3, np3).wait()
                    gather_desc(np3).start()

                @pl.when(t >= 2)
                def _():
                    wb_desc(t - 2, tp).wait()

                compute_chunk(cb, tp)
                wb_desc(t, tp).start()
            return carry

        lax.fori_loop(0, _CHUNKS // 4, quad_body, 0)

        for b in range(2):
            wb_desc(_CHUNKS - 2 + b, b).wait()

    return k(table, idx_flat)


_TI = N // 8
_TJ = N // 128


_TJB = 2


def _tc_body(g_ref, b_ref, out_ref):
    for tjj in range(_TJB):
        out_ref[0, :, pl.ds(tjj * 128, 128)] = (
            jnp.reshape(g_ref[0, :, tjj, :, :], (N, 128))
            + b_ref[0, :, pl.ds(tjj * 128, 128)]
        )


def _tc_assemble_add(g_raw, attn_bias):
    g5 = g_raw.reshape(NUM_HEADS, _TI, _TJ, 8, 128)
    return pl.pallas_call(
        _tc_body,
        grid=(_TJ // _TJB, NUM_HEADS),
        in_specs=[
            pl.BlockSpec(
                (1, _TI, _TJB, 8, 128), lambda tj, h: (h, 0, tj, 0, 0)
            ),
            pl.BlockSpec((1, N, _TJB * 128), lambda tj, h: (0, 0, tj)),
        ],
        out_specs=pl.BlockSpec((1, N, _TJB * 128), lambda tj, h: (h, 0, tj)),
        out_shape=jax.ShapeDtypeStruct((NUM_HEADS, N, N), jnp.float32),
    )(g5, attn_bias)


def kernel(attn_bias, spatial_pos, W):
    idx_tile = (
        spatial_pos.reshape(_TI, 8, _TJ, 128)
        .transpose(0, 2, 1, 3)
        .reshape(NN)
    )
    g_raw = _sc_gather_transpose(idx_tile, W)
    out = _tc_assemble_add(g_raw, attn_bias)
    return out.reshape(1, NUM_HEADS, N, N)


# speedup vs baseline: 1.1218x; 1.0211x over previous
import functools

import jax
import jax.numpy as jnp
from jax import lax
from jax.experimental import pallas as pl
from jax.experimental.pallas import tpu as pltpu
from jax.experimental.pallas import tpu_sc as plsc

NUM_HEADS = 16
N = 2048
NN = N * N

_info = plsc.get_sparse_core_info()
_NC, _NS, _L = _info.num_cores, _info.num_subcores, _info.num_lanes
_NW = _NC * _NS
_B_PER_W = NN // _NW
_C = 1024
_CHUNKS = _B_PER_W // _C
_TPAD = _C + 1
_CPB = 16
_IBC = _CPB * _C
_NBLK = _CHUNKS // _CPB
_UNROLL = 32


def _sc_gather_transpose(idx_flat, table):
    mesh = plsc.VectorSubcoreMesh(core_axis_name="c", subcore_axis_name="s")

    @functools.partial(
        pl.kernel,
        mesh=mesh,
        compiler_params=pltpu.CompilerParams(
            use_tc_tiling_on_sc=False, needs_layout_passes=False
        ),
        out_type=jax.ShapeDtypeStruct((NUM_HEADS, NN), jnp.float32),
        scratch_types=[
            pltpu.VMEM((_C,), jnp.int32),
            pltpu.VMEM((_C,), jnp.int32),
            pltpu.VMEM((_C,), jnp.int32),
            pltpu.VMEM((_C,), jnp.int32),
            pltpu.VMEM((_C, NUM_HEADS), jnp.float32),
            pltpu.VMEM((_C, NUM_HEADS), jnp.float32),
            pltpu.VMEM((_C, NUM_HEADS), jnp.float32),
            pltpu.VMEM((_C, NUM_HEADS), jnp.float32),
            pltpu.VMEM((NUM_HEADS, _TPAD), jnp.float32),
            pltpu.VMEM((NUM_HEADS, _TPAD), jnp.float32),
            pltpu.SemaphoreType.DMA,
            pltpu.SemaphoreType.DMA,
            pltpu.SemaphoreType.DMA,
            pltpu.SemaphoreType.DMA,
            pltpu.SemaphoreType.DMA,
            pltpu.SemaphoreType.DMA,
            pltpu.SemaphoreType.DMA,
            pltpu.SemaphoreType.DMA,
            pltpu.SemaphoreType.DMA,
            pltpu.SemaphoreType.DMA,
        ],
    )
    def k(table_hbm, idx_hbm, out_hbm, i0, i1, i2, i3, r0, r1, r2, r3,
          t0, t1, is0, is1, is2, is3, g0, g1, g2, g3, w0, w1):
        idx_v = (i0, i1, i2, i3)
        rows_v = (r0, r1, r2, r3)
        trans_v = (t0, t1)
        isem = (is0, is1, is2, is3)
        gsem = (g0, g1, g2, g3)
        wsem = (w0, w1)
        wid = lax.axis_index("s") * _NC + lax.axis_index("c")
        wstart = wid * _B_PER_W
        lane = lax.broadcasted_iota(jnp.int32, (_L,), 0)

        def idx_desc(t, p):
            return pltpu.make_async_copy(
                idx_hbm.at[pl.ds(wstart + t * _C, _C)], idx_v[p], isem[p]
            )

        def gather_desc(p):
            return pltpu.make_async_copy(
                table_hbm.at[idx_v[p]], rows_v[p], gsem[p]
            )

        def wb_desc(t, p):
            return pltpu.make_async_copy(
                trans_v[p].at[:, pl.ds(0, _C)],
                out_hbm.at[:, pl.ds(wstart + t * _C, _C)],
                wsem[p],
            )

        def compute_chunk(p, tp):
            rref = rows_v[p]
            tref = trans_v[tp]

            def kb_body(kb, c3):
                base_k = kb * _UNROLL
                colbase = jnp.full((_L,), base_k, jnp.int32)
                for j in range(_UNROLL):
                    v = rref[base_k + j]
                    plsc.store_scatter(tref, [lane, colbase + j], v)
                return c3

            lax.fori_loop(0, _C // _UNROLL, kb_body, 0)

        for p in range(4):
            idx_desc(p, p).start()
        for p in range(3):
            idx_desc(p, p).wait()
            gather_desc(p).start()

        def quad_body(tt, carry):
            for cb in range(4):
                t = tt * 4 + cb
                tp = cb % 2
                np3 = (cb + 3) % 4

                gather_desc(cb).wait()

                @pl.when(t + 4 < _CHUNKS)
                def _():
                    idx_desc(t + 4, cb).start()

                @pl.when(t + 3 < _CHUNKS)
                def _():
                    idx_desc(t + 3, np3).wait()
                    gather_desc(np3).start()

                @pl.when(t >= 2)
                def _():
                    wb_desc(t - 2, tp).wait()

                compute_chunk(cb, tp)
                wb_desc(t, tp).start()
            return carry

        lax.fori_loop(0, _CHUNKS // 4, quad_body, 0)

        for b in range(2):
            wb_desc(_CHUNKS - 2 + b, b).wait()

    return k(table, idx_flat)


_TI = N // 8
_TJ = N // 128


_TJB = 4


def _tc_body(g_ref, b_ref, out_ref):
    for tjj in range(_TJB):
        out_ref[0, :, pl.ds(tjj * 128, 128)] = (
            jnp.reshape(g_ref[0, :, tjj, :, :], (N, 128))
            + b_ref[0, :, pl.ds(tjj * 128, 128)]
        )


def _tc_assemble_add(g_raw, attn_bias):
    g5 = g_raw.reshape(NUM_HEADS, _TI, _TJ, 8, 128)
    return pl.pallas_call(
        _tc_body,
        grid=(_TJ // _TJB, NUM_HEADS),
        in_specs=[
            pl.BlockSpec(
                (1, _TI, _TJB, 8, 128), lambda tj, h: (h, 0, tj, 0, 0)
            ),
            pl.BlockSpec((1, N, _TJB * 128), lambda tj, h: (0, 0, tj)),
        ],
        out_specs=pl.BlockSpec((1, N, _TJB * 128), lambda tj, h: (h, 0, tj)),
        out_shape=jax.ShapeDtypeStruct((NUM_HEADS, N, N), jnp.float32),
    )(g5, attn_bias)


def kernel(attn_bias, spatial_pos, W):
    idx_tile = (
        spatial_pos.reshape(_TI, 8, _TJ, 128)
        .transpose(0, 2, 1, 3)
        .reshape(NN)
    )
    g_raw = _sc_gather_transpose(idx_tile, W)
    out = _tc_assemble_add(g_raw, attn_bias)
    return out.reshape(1, NUM_HEADS, N, N)


# speedup vs baseline: 1.2120x; 1.0804x over previous
import functools

import jax
import jax.numpy as jnp
from jax import lax
from jax.experimental import pallas as pl
from jax.experimental.pallas import tpu as pltpu
from jax.experimental.pallas import tpu_sc as plsc

NUM_HEADS = 16
N = 2048
NN = N * N

_info = plsc.get_sparse_core_info()
_NC, _NS, _L = _info.num_cores, _info.num_subcores, _info.num_lanes
_NW = _NC * _NS
_B_PER_W = NN // _NW
_C = 1024
_CHUNKS = _B_PER_W // _C
_TPAD = _C + 1
_CPB = 16
_IBC = _CPB * _C
_NBLK = _CHUNKS // _CPB
_UNROLL = 32


_S = 2
_NNH = NN // _S
_BW = _NNH // _NW
_CH = _BW // _C


def _sc_gather_transpose(idx_flat, table, half):
    mesh = plsc.VectorSubcoreMesh(core_axis_name="c", subcore_axis_name="s")

    @functools.partial(
        pl.kernel,
        mesh=mesh,
        compiler_params=pltpu.CompilerParams(
            use_tc_tiling_on_sc=False, needs_layout_passes=False
        ),
        out_type=jax.ShapeDtypeStruct((NUM_HEADS, _NNH), jnp.float32),
        scratch_types=[
            pltpu.VMEM((_C,), jnp.int32),
            pltpu.VMEM((_C,), jnp.int32),
            pltpu.VMEM((_C,), jnp.int32),
            pltpu.VMEM((_C,), jnp.int32),
            pltpu.VMEM((_C, NUM_HEADS), jnp.float32),
            pltpu.VMEM((_C, NUM_HEADS), jnp.float32),
            pltpu.VMEM((_C, NUM_HEADS), jnp.float32),
            pltpu.VMEM((_C, NUM_HEADS), jnp.float32),
            pltpu.VMEM((NUM_HEADS, _TPAD), jnp.float32),
            pltpu.VMEM((NUM_HEADS, _TPAD), jnp.float32),
            pltpu.SemaphoreType.DMA,
            pltpu.SemaphoreType.DMA,
            pltpu.SemaphoreType.DMA,
            pltpu.SemaphoreType.DMA,
            pltpu.SemaphoreType.DMA,
            pltpu.SemaphoreType.DMA,
            pltpu.SemaphoreType.DMA,
            pltpu.SemaphoreType.DMA,
            pltpu.SemaphoreType.DMA,
            pltpu.SemaphoreType.DMA,
        ],
    )
    def k(table_hbm, idx_hbm, out_hbm, i0, i1, i2, i3, r0, r1, r2, r3,
          t0, t1, is0, is1, is2, is3, g0, g1, g2, g3, w0, w1):
        idx_v = (i0, i1, i2, i3)
        rows_v = (r0, r1, r2, r3)
        trans_v = (t0, t1)
        isem = (is0, is1, is2, is3)
        gsem = (g0, g1, g2, g3)
        wsem = (w0, w1)
        wid = lax.axis_index("s") * _NC + lax.axis_index("c")
        lstart = wid * _BW
        gstart = half * _NNH + lstart
        lane = lax.broadcasted_iota(jnp.int32, (_L,), 0)

        def idx_desc(t, p):
            return pltpu.make_async_copy(
                idx_hbm.at[pl.ds(gstart + t * _C, _C)], idx_v[p], isem[p]
            )

        def gather_desc(p):
            return pltpu.make_async_copy(
                table_hbm.at[idx_v[p]], rows_v[p], gsem[p]
            )

        def wb_desc(t, p):
            return pltpu.make_async_copy(
                trans_v[p].at[:, pl.ds(0, _C)],
                out_hbm.at[:, pl.ds(lstart + t * _C, _C)],
                wsem[p],
            )

        def compute_chunk(p, tp):
            rref = rows_v[p]
            tref = trans_v[tp]

            def kb_body(kb, c3):
                base_k = kb * _UNROLL
                colbase = jnp.full((_L,), base_k, jnp.int32)
                for j in range(_UNROLL):
                    v = rref[base_k + j]
                    plsc.store_scatter(tref, [lane, colbase + j], v)
                return c3

            lax.fori_loop(0, _C // _UNROLL, kb_body, 0)

        for p in range(4):
            idx_desc(p, p).start()
        for p in range(3):
            idx_desc(p, p).wait()
            gather_desc(p).start()

        def quad_body(tt, carry):
            for cb in range(4):
                t = tt * 4 + cb
                tp = cb % 2
                np3 = (cb + 3) % 4

                gather_desc(cb).wait()

                @pl.when(t + 4 < _CH)
                def _():
                    idx_desc(t + 4, cb).start()

                @pl.when(t + 3 < _CH)
                def _():
                    idx_desc(t + 3, np3).wait()
                    gather_desc(np3).start()

                @pl.when(t >= 2)
                def _():
                    wb_desc(t - 2, tp).wait()

                compute_chunk(cb, tp)
                wb_desc(t, tp).start()
            return carry

        lax.fori_loop(0, _CH // 4, quad_body, 0)

        for b in range(2):
            wb_desc(_CH - 2 + b, b).wait()

    return k(table, idx_flat)


_TI = N // 8
_TJ = N // 128


_TJB = 4


_NH = N // _S


def _tc_body_first(g_ref, b_ref, out_ref):
    for tjj in range(_TJB):
        out_ref[0, :, pl.ds(tjj * 128, 128)] = (
            jnp.reshape(g_ref[0, :, tjj, :, :], (_NH, 128))
            + b_ref[0, :, pl.ds(tjj * 128, 128)]
        )


def _tc_body_next(g_ref, b_ref, prev_ref, out_ref):
    del prev_ref
    _tc_body_first(g_ref, b_ref, out_ref)


def _tc_assemble_add(g_half, attn_bias, half, prev):
    g5 = g_half.reshape(NUM_HEADS, _TI // _S, _TJ, 8, 128)
    gspec = pl.BlockSpec(
        (1, _TI // _S, _TJB, 8, 128), lambda tj, h: (h, 0, tj, 0, 0)
    )
    bspec = pl.BlockSpec(
        (1, _NH, _TJB * 128), lambda tj, h: (0, half, tj)
    )
    ospec = pl.BlockSpec(
        (1, _NH, _TJB * 128), lambda tj, h: (h, half, tj)
    )
    out_shape = jax.ShapeDtypeStruct((NUM_HEADS, N, N), jnp.float32)
    if prev is None:
        return pl.pallas_call(
            _tc_body_first,
            grid=(_TJ // _TJB, NUM_HEADS),
            in_specs=[gspec, bspec],
            out_specs=ospec,
            out_shape=out_shape,
        )(g5, attn_bias)
    return pl.pallas_call(
        _tc_body_next,
        grid=(_TJ // _TJB, NUM_HEADS),
        in_specs=[gspec, bspec, pl.BlockSpec(memory_space=pl.ANY)],
        out_specs=ospec,
        out_shape=out_shape,
        input_output_aliases={2: 0},
    )(g5, attn_bias, prev)


def kernel(attn_bias, spatial_pos, W):
    idx_tile = (
        spatial_pos.reshape(_TI, 8, _TJ, 128)
        .transpose(0, 2, 1, 3)
        .reshape(NN)
    )
    g_halves = [_sc_gather_transpose(idx_tile, W, s) for s in range(_S)]
    out = None
    for s in range(_S):
        out = _tc_assemble_add(g_halves[s], attn_bias, s, out)
    return out.reshape(1, NUM_HEADS, N, N)


# speedup vs baseline: 1.2276x; 1.0129x over previous
import functools

import jax
import jax.numpy as jnp
from jax import lax
from jax.experimental import pallas as pl
from jax.experimental.pallas import tpu as pltpu
from jax.experimental.pallas import tpu_sc as plsc

NUM_HEADS = 16
N = 2048
NN = N * N

_info = plsc.get_sparse_core_info()
_NC, _NS, _L = _info.num_cores, _info.num_subcores, _info.num_lanes
_NW = _NC * _NS
_B_PER_W = NN // _NW
_C = 1024
_CHUNKS = _B_PER_W // _C
_TPAD = _C + 1
_CPB = 16
_IBC = _CPB * _C
_NBLK = _CHUNKS // _CPB
_UNROLL = 32


_S = 4
_NNH = NN // _S
_BW = _NNH // _NW
_CH = _BW // _C


def _sc_gather_transpose(idx_flat, table, half):
    mesh = plsc.VectorSubcoreMesh(core_axis_name="c", subcore_axis_name="s")

    @functools.partial(
        pl.kernel,
        mesh=mesh,
        compiler_params=pltpu.CompilerParams(
            use_tc_tiling_on_sc=False, needs_layout_passes=False
        ),
        out_type=jax.ShapeDtypeStruct((NUM_HEADS, _NNH), jnp.float32),
        scratch_types=[
            pltpu.VMEM((_C,), jnp.int32),
            pltpu.VMEM((_C,), jnp.int32),
            pltpu.VMEM((_C,), jnp.int32),
            pltpu.VMEM((_C,), jnp.int32),
            pltpu.VMEM((_C, NUM_HEADS), jnp.float32),
            pltpu.VMEM((_C, NUM_HEADS), jnp.float32),
            pltpu.VMEM((_C, NUM_HEADS), jnp.float32),
            pltpu.VMEM((_C, NUM_HEADS), jnp.float32),
            pltpu.VMEM((NUM_HEADS, _TPAD), jnp.float32),
            pltpu.VMEM((NUM_HEADS, _TPAD), jnp.float32),
            pltpu.SemaphoreType.DMA,
            pltpu.SemaphoreType.DMA,
            pltpu.SemaphoreType.DMA,
            pltpu.SemaphoreType.DMA,
            pltpu.SemaphoreType.DMA,
            pltpu.SemaphoreType.DMA,
            pltpu.SemaphoreType.DMA,
            pltpu.SemaphoreType.DMA,
            pltpu.SemaphoreType.DMA,
            pltpu.SemaphoreType.DMA,
        ],
    )
    def k(table_hbm, idx_hbm, out_hbm, i0, i1, i2, i3, r0, r1, r2, r3,
          t0, t1, is0, is1, is2, is3, g0, g1, g2, g3, w0, w1):
        idx_v = (i0, i1, i2, i3)
        rows_v = (r0, r1, r2, r3)
        trans_v = (t0, t1)
        isem = (is0, is1, is2, is3)
        gsem = (g0, g1, g2, g3)
        wsem = (w0, w1)
        wid = lax.axis_index("s") * _NC + lax.axis_index("c")
        lstart = wid * _BW
        gstart = half * _NNH + lstart
        lane = lax.broadcasted_iota(jnp.int32, (_L,), 0)

        def idx_desc(t, p):
            return pltpu.make_async_copy(
                idx_hbm.at[pl.ds(gstart + t * _C, _C)], idx_v[p], isem[p]
            )

        def gather_desc(p):
            return pltpu.make_async_copy(
                table_hbm.at[idx_v[p]], rows_v[p], gsem[p]
            )

        def wb_desc(t, p):
            return pltpu.make_async_copy(
                trans_v[p].at[:, pl.ds(0, _C)],
                out_hbm.at[:, pl.ds(lstart + t * _C, _C)],
                wsem[p],
            )

        def compute_chunk(p, tp):
            rref = rows_v[p]
            tref = trans_v[tp]

            def kb_body(kb, c3):
                base_k = kb * _UNROLL
                colbase = jnp.full((_L,), base_k, jnp.int32)
                for j in range(_UNROLL):
                    v = rref[base_k + j]
                    plsc.store_scatter(tref, [lane, colbase + j], v)
                return c3

            lax.fori_loop(0, _C // _UNROLL, kb_body, 0)

        for p in range(4):
            idx_desc(p, p).start()
        for p in range(3):
            idx_desc(p, p).wait()
            gather_desc(p).start()

        def quad_body(tt, carry):
            for cb in range(4):
                t = tt * 4 + cb
                tp = cb % 2
                np3 = (cb + 3) % 4

                gather_desc(cb).wait()

                @pl.when(t + 4 < _CH)
                def _():
                    idx_desc(t + 4, cb).start()

                @pl.when(t + 3 < _CH)
                def _():
                    idx_desc(t + 3, np3).wait()
                    gather_desc(np3).start()

                @pl.when(t >= 2)
                def _():
                    wb_desc(t - 2, tp).wait()

                compute_chunk(cb, tp)
                wb_desc(t, tp).start()
            return carry

        lax.fori_loop(0, _CH // 4, quad_body, 0)

        for b in range(2):
            wb_desc(_CH - 2 + b, b).wait()

    return k(table, idx_flat)


_TI = N // 8
_TJ = N // 128


_TJB = 4


_NH = N // _S


def _tc_body_first(g_ref, b_ref, out_ref):
    for tjj in range(_TJB):
        out_ref[0, :, pl.ds(tjj * 128, 128)] = (
            jnp.reshape(g_ref[0, :, tjj, :, :], (_NH, 128))
            + b_ref[0, :, pl.ds(tjj * 128, 128)]
        )


def _tc_body_next(g_ref, b_ref, prev_ref, out_ref):
    del prev_ref
    _tc_body_first(g_ref, b_ref, out_ref)


def _tc_assemble_add(g_half, attn_bias, half, prev):
    g5 = g_half.reshape(NUM_HEADS, _TI // _S, _TJ, 8, 128)
    gspec = pl.BlockSpec(
        (1, _TI // _S, _TJB, 8, 128), lambda tj, h: (h, 0, tj, 0, 0)
    )
    bspec = pl.BlockSpec(
        (1, _NH, _TJB * 128), lambda tj, h: (0, half, tj)
    )
    ospec = pl.BlockSpec(
        (1, _NH, _TJB * 128), lambda tj, h: (h, half, tj)
    )
    out_shape = jax.ShapeDtypeStruct((NUM_HEADS, N, N), jnp.float32)
    if prev is None:
        return pl.pallas_call(
            _tc_body_first,
            grid=(_TJ // _TJB, NUM_HEADS),
            in_specs=[gspec, bspec],
            out_specs=ospec,
            out_shape=out_shape,
        )(g5, attn_bias)
    return pl.pallas_call(
        _tc_body_next,
        grid=(_TJ // _TJB, NUM_HEADS),
        in_specs=[gspec, bspec, pl.BlockSpec(memory_space=pl.ANY)],
        out_specs=ospec,
        out_shape=out_shape,
        input_output_aliases={2: 0},
    )(g5, attn_bias, prev)


def kernel(attn_bias, spatial_pos, W):
    idx_tile = (
        spatial_pos.reshape(_TI, 8, _TJ, 128)
        .transpose(0, 2, 1, 3)
        .reshape(NN)
    )
    g_halves = [_sc_gather_transpose(idx_tile, W, s) for s in range(_S)]
    out = None
    for s in range(_S):
        out = _tc_assemble_add(g_halves[s], attn_bias, s, out)
    return out.reshape(1, NUM_HEADS, N, N)
